# baseline (speedup 1.0000x reference)
import functools

import jax
import jax.numpy as jnp
from jax import lax
from jax.experimental import pallas as pl
from jax.experimental.pallas import tpu as pltpu
from jax.experimental.pallas import tpu_sc as plsc

BATCH = 16384
EMBED_DIM = 32
NUM_CORES = 2
NUM_SUBCORES = 16
LANES = 16
NUM_WORKERS = NUM_CORES * NUM_SUBCORES
BPW = BATCH // NUM_WORKERS
GROUPS = BPW // LANES


def _gmf_sc(user, item, embed_u, embed_i, params):
    mesh = plsc.VectorSubcoreMesh(core_axis_name="c", subcore_axis_name="s")

    @functools.partial(
        pl.kernel,
        out_type=jax.ShapeDtypeStruct((BATCH,), jnp.float32),
        mesh=mesh,
        scratch_types=[
            pltpu.VMEM((BPW,), jnp.int32),
            pltpu.VMEM((BPW,), jnp.int32),
            pltpu.VMEM((BPW, EMBED_DIM), jnp.float32),
            pltpu.VMEM((BPW, EMBED_DIM), jnp.float32),
            pltpu.VMEM((48,), jnp.float32),
            pltpu.VMEM((BPW,), jnp.float32),
            pltpu.SemaphoreType.DMA,
            pltpu.SemaphoreType.DMA,
        ],
        compiler_params=pltpu.CompilerParams(
            needs_layout_passes=False, use_tc_tiling_on_sc=False),
    )
    def k(u_hbm, i_hbm, eu_hbm, ei_hbm, p_hbm, o_hbm,
          uix, iix, urows, irows, pv, ov, sem_u, sem_i):
        wid = lax.axis_index("s") * NUM_CORES + lax.axis_index("c")
        base = wid * BPW
        pltpu.sync_copy(u_hbm.at[pl.ds(base, BPW)], uix)
        pltpu.sync_copy(i_hbm.at[pl.ds(base, BPW)], iix)
        pltpu.sync_copy(p_hbm, pv)
        cu = pltpu.async_copy(eu_hbm.at[uix], urows, sem_u)
        ci = pltpu.async_copy(ei_hbm.at[iix], irows, sem_i)
        cu.wait()
        ci.wait()

        lane = lax.iota(jnp.int32, LANES)
        h0 = pv[pl.ds(0, LANES)]
        h1 = pv[pl.ds(LANES, LANES)]
        hb = pv[pl.ds(2 * LANES, LANES)][0]

        @pl.loop(0, BPW, step=LANES)
        def _(g):
            rows = lane + g
            acc = jnp.zeros((LANES,), jnp.float32)
            for d in range(EMBED_DIM):
                cols = jnp.full((LANES,), d, jnp.int32)
                uc = plsc.load_gather(urows, [rows, cols])
                ic = plsc.load_gather(irows, [rows, cols])
                hd = h0[d] if d < LANES else h1[d - LANES]
                acc = acc + uc * ic * hd
            ov[pl.ds(g, LANES)] = 1.0 / (1.0 + jnp.exp(-(acc + hb)))

        pltpu.sync_copy(ov, o_hbm.at[pl.ds(base, BPW)])

    return k(user, item, embed_u, embed_i, params)


def kernel(user, item, binary, embed_u, embed_i, H_w, H_b):
    del binary
    user = user.astype(jnp.int32)
    item = item.astype(jnp.int32)
    params = jnp.concatenate(
        [H_w.reshape(-1), H_b.reshape(-1),
         jnp.zeros((15,), jnp.float32)]).astype(jnp.float32)
    out = _gmf_sc(user, item, embed_u, embed_i, params)
    return out.reshape(BATCH, 1)


# speedup vs baseline: 3.1238x; 3.1238x over previous
import functools

import jax
import jax.numpy as jnp
from jax import lax
from jax.experimental import pallas as pl
from jax.experimental.pallas import tpu as pltpu
from jax.experimental.pallas import tpu_sc as plsc

BATCH = 16384
EMBED_DIM = 32
NROWS = 1000000
NUM_CORES = 2
NUM_SUBCORES = 16
LANES = 16
NUM_WORKERS = NUM_CORES * NUM_SUBCORES
BPW = BATCH // NUM_WORKERS
OCT = 8
NOCT = BPW // OCT
MAX_BASE = (NROWS - 128) // 128 * 128
TAIL = NROWS - 128
TAIL_CUT = MAX_BASE + 128


def _gmf_sc(user, item, embed_u_t, embed_i_t, u_tail, i_tail, params):
    mesh = plsc.VectorSubcoreMesh(core_axis_name="c", subcore_axis_name="s")

    slot_types = [pltpu.VMEM((EMBED_DIM, 128), jnp.float32)
                  for _ in range(2 * OCT)]

    @functools.partial(
        pl.kernel,
        out_type=jax.ShapeDtypeStruct((BATCH,), jnp.float32),
        mesh=mesh,
        scratch_types=[
            pltpu.VMEM((BPW + 16,), jnp.int32),
            pltpu.VMEM((BPW + 16,), jnp.int32),
            pltpu.VMEM((EMBED_DIM, 128), jnp.float32),
            pltpu.VMEM((EMBED_DIM, 128), jnp.float32),
            pltpu.VMEM((48,), jnp.float32),
            pltpu.VMEM((256,), jnp.float32),
            pltpu.VMEM((BPW + 8,), jnp.float32),
            pltpu.SemaphoreType.DMA,
            pltpu.SemaphoreType.DMA,
        ] + slot_types,
        compiler_params=pltpu.CompilerParams(
            use_tc_tiling_on_sc=True, needs_layout_passes=False),
    )
    def k(u_hbm, i_hbm, eu_hbm, ei_hbm, ut_hbm, it_hbm, p_hbm, o_hbm,
          uix, iix, utail, itail, pv, stage, ov, sem_u, sem_i, *slots):
        uslots = slots[:OCT]
        islots = slots[OCT:]
        wid = lax.axis_index("s") * NUM_CORES + lax.axis_index("c")
        base = wid * BPW
        pltpu.sync_copy(u_hbm.at[pl.ds(base, BPW)], uix.at[pl.ds(0, BPW)])
        pltpu.sync_copy(i_hbm.at[pl.ds(base, BPW)], iix.at[pl.ds(0, BPW)])
        pltpu.sync_copy(p_hbm, pv)
        pltpu.sync_copy(ut_hbm, utail)
        pltpu.sync_copy(it_hbm, itail)

        h0 = pv[pl.ds(0, LANES)]
        h1 = pv[pl.ds(LANES, LANES)]
        hb = pv[pl.ds(2 * LANES, LANES)][0]
        rows16 = lax.iota(jnp.int32, LANES)
        lane16x16 = rows16 * LANES

        def fire(o):
            uch = jnp.clip(uix[pl.ds(o * OCT, LANES)], 0, NROWS - 1)
            ich = jnp.clip(iix[pl.ds(o * OCT, LANES)], 0, NROWS - 1)
            for j in range(OCT):
                ub = jnp.minimum(uch[j] & -128, MAX_BASE)
                ib = jnp.minimum(ich[j] & -128, MAX_BASE)
                pltpu.async_copy(
                    eu_hbm.at[:, pl.ds(pl.multiple_of(ub, 128), 128)],
                    uslots[j], sem_u)
                pltpu.async_copy(
                    ei_hbm.at[:, pl.ds(pl.multiple_of(ib, 128), 128)],
                    islots[j], sem_i)

        def drain():
            for j in range(OCT):
                pltpu.make_async_copy(
                    eu_hbm.at[:, pl.ds(0, 128)], uslots[j], sem_u).wait()
                pltpu.make_async_copy(
                    ei_hbm.at[:, pl.ds(0, 128)], islots[j], sem_i).wait()

        def compute(o):
            uch = jnp.clip(uix[pl.ds(o * OCT, LANES)], 0, NROWS - 1)
            ich = jnp.clip(iix[pl.ds(o * OCT, LANES)], 0, NROWS - 1)
            for j in range(OCT):
                ur, ir = uch[j], ich[j]
                ul = jnp.minimum(ur - jnp.minimum(ur & -128, MAX_BASE), 127)
                il = jnp.minimum(ir - jnp.minimum(ir & -128, MAX_BASE), 127)
                ucol = jnp.full((LANES,), ul, jnp.int32)
                icol = jnp.full((LANES,), il, jnp.int32)
                u0 = plsc.load_gather(uslots[j], [rows16, ucol])
                u1 = plsc.load_gather(uslots[j], [rows16 + LANES, ucol])
                i0 = plsc.load_gather(islots[j], [rows16, icol])
                i1 = plsc.load_gather(islots[j], [rows16 + LANES, icol])
                utc = jnp.full((LANES,), jnp.clip(ur - TAIL, 0, 127),
                               jnp.int32)
                itc = jnp.full((LANES,), jnp.clip(ir - TAIL, 0, 127),
                               jnp.int32)
                um = jnp.full((LANES,), ur >= TAIL_CUT)
                im = jnp.full((LANES,), ir >= TAIL_CUT)
                u0 = jnp.where(um, plsc.load_gather(utail, [rows16, utc]), u0)
                u1 = jnp.where(
                    um, plsc.load_gather(utail, [rows16 + LANES, utc]), u1)
                i0 = jnp.where(im, plsc.load_gather(itail, [rows16, itc]), i0)
                i1 = jnp.where(
                    im, plsc.load_gather(itail, [rows16 + LANES, itc]), i1)
                stage[pl.ds(j * LANES, LANES)] = u0 * i0 * h0 + u1 * i1 * h1
            acc = jnp.zeros((LANES,), jnp.float32)
            for m in range(LANES):
                acc = acc + plsc.load_gather(stage, [lane16x16 + m])
            ov[pl.ds(o * OCT, LANES)] = 1.0 / (1.0 + jnp.exp(-(acc + hb)))

        fire(0)

        @pl.loop(0, NOCT - 1)
        def _(o):
            drain()
            compute(o)
            fire(o + 1)

        drain()
        compute(NOCT - 1)

        pltpu.sync_copy(ov.at[pl.ds(0, BPW)], o_hbm.at[pl.ds(base, BPW)])

    return k(user, item, embed_u_t, embed_i_t, u_tail, i_tail, params)


def kernel(user, item, binary, embed_u, embed_i, H_w, H_b):
    del binary
    user = user.astype(jnp.int32)
    item = item.astype(jnp.int32)
    params = jnp.concatenate(
        [H_w.reshape(-1), H_b.reshape(-1),
         jnp.zeros((15,), jnp.float32)]).astype(jnp.float32)
    u_tail = embed_u[TAIL:].T
    i_tail = embed_i[TAIL:].T
    out = _gmf_sc(user, item, embed_u.T, embed_i.T, u_tail, i_tail, params)
    return out.reshape(BATCH, 1)


# speedup vs baseline: 4.4068x; 1.4107x over previous
import functools

import jax
import jax.numpy as jnp
from jax import lax
from jax.experimental import pallas as pl
from jax.experimental.pallas import tpu as pltpu
from jax.experimental.pallas import tpu_sc as plsc

BATCH = 16384
EMBED_DIM = 32
NROWS = 1000000
NUM_CORES = 2
NUM_SUBCORES = 16
LANES = 16
NUM_WORKERS = NUM_CORES * NUM_SUBCORES
BPW = BATCH // NUM_WORKERS
OCT = 8
NOCT = BPW // OCT
MAX_BASE = (NROWS - 128) // 128 * 128
TAIL = NROWS - 128
TAIL_CUT = MAX_BASE + 128


def _gmf_sc(user, item, embed_u_t, embed_i_t, u_tail, i_tail, params):
    mesh = plsc.VectorSubcoreMesh(core_axis_name="c", subcore_axis_name="s")

    slot_types = [pltpu.VMEM((EMBED_DIM, 128), jnp.float32)
                  for _ in range(2 * OCT)]

    @functools.partial(
        pl.kernel,
        out_type=jax.ShapeDtypeStruct((BATCH,), jnp.float32),
        mesh=mesh,
        scratch_types=[
            pltpu.VMEM((BPW + 16,), jnp.int32),
            pltpu.VMEM((BPW + 16,), jnp.int32),
            pltpu.VMEM((EMBED_DIM, 128), jnp.float32),
            pltpu.VMEM((EMBED_DIM, 128), jnp.float32),
            pltpu.VMEM((48,), jnp.float32),
            pltpu.VMEM((256,), jnp.float32),
            pltpu.VMEM((BPW + 8,), jnp.float32),
            pltpu.SemaphoreType.DMA,
            pltpu.SemaphoreType.DMA,
        ] + slot_types,
        compiler_params=pltpu.CompilerParams(
            use_tc_tiling_on_sc=True, needs_layout_passes=False),
    )
    def k(u_hbm, i_hbm, eu_hbm, ei_hbm, ut_hbm, it_hbm, p_hbm, o_hbm,
          uix, iix, utail, itail, pv, stage, ov, sem_u, sem_i, *slots):
        uslots = slots[:OCT]
        islots = slots[OCT:]
        wid = lax.axis_index("s") * NUM_CORES + lax.axis_index("c")
        base = wid * BPW
        pltpu.sync_copy(u_hbm.at[pl.ds(base, BPW)], uix.at[pl.ds(0, BPW)])
        pltpu.sync_copy(i_hbm.at[pl.ds(base, BPW)], iix.at[pl.ds(0, BPW)])
        pltpu.sync_copy(p_hbm, pv)
        pltpu.sync_copy(ut_hbm, utail)
        pltpu.sync_copy(it_hbm, itail)

        h0 = pv[pl.ds(0, LANES)]
        h1 = pv[pl.ds(LANES, LANES)]
        hb = pv[pl.ds(2 * LANES, LANES)][0]
        rows16 = lax.iota(jnp.int32, LANES)
        lane16x16 = rows16 * LANES

        def fire_row(j, ur, ir):
            ub = jnp.minimum(ur & -128, MAX_BASE)
            ib = jnp.minimum(ir & -128, MAX_BASE)
            pltpu.async_copy(
                eu_hbm.at[:, pl.ds(pl.multiple_of(ub, 128), 128)],
                uslots[j], sem_u)
            pltpu.async_copy(
                ei_hbm.at[:, pl.ds(pl.multiple_of(ib, 128), 128)],
                islots[j], sem_i)

        def fire(o):
            uch = jnp.clip(uix[pl.ds(o * OCT, LANES)], 0, NROWS - 1)
            ich = jnp.clip(iix[pl.ds(o * OCT, LANES)], 0, NROWS - 1)
            for j in range(OCT):
                fire_row(j, uch[j], ich[j])

        def drain_row(j):
            pltpu.make_async_copy(
                eu_hbm.at[:, pl.ds(0, 128)], uslots[j], sem_u).wait()
            pltpu.make_async_copy(
                ei_hbm.at[:, pl.ds(0, 128)], islots[j], sem_i).wait()

        def compute_row(j, ur, ir):
            ul = jnp.minimum(ur - jnp.minimum(ur & -128, MAX_BASE), 127)
            il = jnp.minimum(ir - jnp.minimum(ir & -128, MAX_BASE), 127)
            ucol = jnp.full((LANES,), ul, jnp.int32)
            icol = jnp.full((LANES,), il, jnp.int32)
            u0 = plsc.load_gather(uslots[j], [rows16, ucol])
            u1 = plsc.load_gather(uslots[j], [rows16 + LANES, ucol])
            i0 = plsc.load_gather(islots[j], [rows16, icol])
            i1 = plsc.load_gather(islots[j], [rows16 + LANES, icol])
            utc = jnp.full((LANES,), jnp.clip(ur - TAIL, 0, 127), jnp.int32)
            itc = jnp.full((LANES,), jnp.clip(ir - TAIL, 0, 127), jnp.int32)
            um = jnp.full((LANES,), ur >= TAIL_CUT)
            im = jnp.full((LANES,), ir >= TAIL_CUT)
            u0 = jnp.where(um, plsc.load_gather(utail, [rows16, utc]), u0)
            u1 = jnp.where(
                um, plsc.load_gather(utail, [rows16 + LANES, utc]), u1)
            i0 = jnp.where(im, plsc.load_gather(itail, [rows16, itc]), i0)
            i1 = jnp.where(
                im, plsc.load_gather(itail, [rows16 + LANES, itc]), i1)
            stage[pl.ds(j * LANES, LANES)] = u0 * i0 * h0 + u1 * i1 * h1

        def reduce_store(o):
            acc = jnp.zeros((LANES,), jnp.float32)
            for m in range(LANES):
                acc = acc + plsc.load_gather(stage, [lane16x16 + m])
            ov[pl.ds(o * OCT, LANES)] = 1.0 / (1.0 + jnp.exp(-(acc + hb)))

        fire(0)

        @pl.loop(0, NOCT - 1)
        def _(o):
            uch = jnp.clip(uix[pl.ds(o * OCT, LANES)], 0, NROWS - 1)
            ich = jnp.clip(iix[pl.ds(o * OCT, LANES)], 0, NROWS - 1)
            nuch = jnp.clip(uix[pl.ds(o * OCT + OCT, LANES)], 0, NROWS - 1)
            nich = jnp.clip(iix[pl.ds(o * OCT + OCT, LANES)], 0, NROWS - 1)
            for j in range(OCT):
                drain_row(j)
                compute_row(j, uch[j], ich[j])
                fire_row(j, nuch[j], nich[j])
            reduce_store(o)

        uch = jnp.clip(uix[pl.ds((NOCT - 1) * OCT, LANES)], 0, NROWS - 1)
        ich = jnp.clip(iix[pl.ds((NOCT - 1) * OCT, LANES)], 0, NROWS - 1)
        for j in range(OCT):
            drain_row(j)
            compute_row(j, uch[j], ich[j])
        reduce_store(NOCT - 1)

        pltpu.sync_copy(ov.at[pl.ds(0, BPW)], o_hbm.at[pl.ds(base, BPW)])

    return k(user, item, embed_u_t, embed_i_t, u_tail, i_tail, params)


def kernel(user, item, binary, embed_u, embed_i, H_w, H_b):
    del binary
    user = user.astype(jnp.int32)
    item = item.astype(jnp.int32)
    params = jnp.concatenate(
        [H_w.reshape(-1), H_b.reshape(-1),
         jnp.zeros((15,), jnp.float32)]).astype(jnp.float32)
    u_tail = embed_u[TAIL:].T
    i_tail = embed_i[TAIL:].T
    out = _gmf_sc(user, item, embed_u.T, embed_i.T, u_tail, i_tail, params)
    return out.reshape(BATCH, 1)
